# baseline (speedup 1.0000x reference)
import jax
import jax.numpy as jnp
from jax.experimental import pallas as pl
from jax.experimental.pallas import tpu as pltpu

_K = 32
_BLOCK = 2000

_DN = (((1,), (1,)), ((), ()))


def _mlp_kernel(a_ref, w1_ref, we1_ref, we2_ref, wd1_ref, wd2_ref, c_ref,
                out_ref):
    f32 = jnp.float32
    b1 = c_ref[0:1, 0:256]
    be1 = c_ref[0:1, 256:384]
    be2 = c_ref[0:1, 512:518]
    bd1 = c_ref[0:1, 640:768]
    bd2 = c_ref[0:1, 768:1024]
    bo = c_ref[0:1, 1024:1025]
    wo = c_ref[1:2, 0:256]

    def lin(v, w_ref, b):
        return jax.lax.dot_general(v, w_ref[...], _DN,
                                   preferred_element_type=f32) + b

    a = a_ref[...]
    h = jnp.tanh(lin(a, w1_ref, b1))
    h = jnp.tanh(lin(h, we1_ref, be1))
    e = jnp.tanh(lin(h, we2_ref, be2))
    h = jnp.tanh(lin(e, wd1_ref, bd1))
    d = lin(h, wd2_ref, bd2)
    t = jnp.tanh(d)
    o = jnp.sum(t * wo, axis=1, keepdims=True) + bo
    out_ref[...] = jax.nn.sigmoid(o)


def kernel(x, edge_index, edge_attr, W_ih, W_hh, b_ih, b_hh, W1, b1, Wg, bg,
           We1, be1, We2, be2, Wd1, bd1, Wd2, bd2, Wo, bo):
    a = edge_attr.reshape(-1, _K)
    rows = a.shape[0]

    canvas = jnp.zeros((8, 1152), jnp.float32)
    canvas = (canvas.at[0, 0:256].set(b1)
                    .at[0, 256:384].set(be1)
                    .at[0, 512:518].set(be2)
                    .at[0, 640:768].set(bd1)
                    .at[0, 768:1024].set(bd2)
                    .at[0, 1024:1025].set(bo)
                    .at[1, 0:256].set(Wo[0]))

    def full(shape):
        return pl.BlockSpec(shape, lambda i: (0,) * len(shape))

    out = pl.pallas_call(
        _mlp_kernel,
        grid=(rows // _BLOCK,),
        in_specs=[
            pl.BlockSpec((_BLOCK, _K), lambda i: (i, 0)),
            full((256, _K)),
            full((128, 256)),
            full((6, 128)),
            full((128, 6)),
            full((256, 128)),
            full((8, 1152)),
        ],
        out_specs=pl.BlockSpec((_BLOCK, 1), lambda i: (i, 0)),
        out_shape=jax.ShapeDtypeStruct((rows, 1), jnp.float32),
        compiler_params=pltpu.CompilerParams(
            dimension_semantics=("parallel",)),
    )(a, W1, We1, We2, Wd1, Wd2, canvas)
    return out[:, 0]


# speedup vs baseline: 1.8395x; 1.8395x over previous
import jax
import jax.numpy as jnp
from jax.experimental import pallas as pl

_K = 32
_BLOCK = 2000

_DN = (((1,), (1,)), ((), ()))


def _mlp_kernel(a_ref, w1_ref, b1_ref, we1_ref, be1_ref, we2_ref, be2_ref,
                wd1_ref, bd1_ref, wd2_ref, bd2_ref, wo_ref, bo_ref, out_ref):
    f32 = jnp.float32

    def lin(v, w_ref, b_ref):
        return jax.lax.dot_general(v, w_ref[...], _DN,
                                   preferred_element_type=f32) + b_ref[...]

    a = a_ref[...]
    h = jnp.tanh(lin(a, w1_ref, b1_ref))
    h = jnp.tanh(lin(h, we1_ref, be1_ref))
    e = jnp.tanh(lin(h, we2_ref, be2_ref))
    h = jnp.tanh(lin(e, wd1_ref, bd1_ref))
    d = lin(h, wd2_ref, bd2_ref)
    t = jnp.tanh(d)
    o = jnp.sum(t * wo_ref[...], axis=1, keepdims=True) + bo_ref[...]
    out_ref[...] = jax.nn.sigmoid(o)


def kernel(x, edge_index, edge_attr, W_ih, W_hh, b_ih, b_hh, W1, b1, Wg, bg,
           We1, be1, We2, be2, Wd1, bd1, Wd2, bd2, Wo, bo):
    a = edge_attr.reshape(-1, _K)
    rows = a.shape[0]

    def full(shape):
        return pl.BlockSpec(shape, lambda i: (0,) * len(shape))

    out = pl.pallas_call(
        _mlp_kernel,
        grid=(rows // _BLOCK,),
        in_specs=[
            pl.BlockSpec((_BLOCK, _K), lambda i: (i, 0)),
            full((256, _K)), full((1, 256)),
            full((128, 256)), full((1, 128)),
            full((6, 128)), full((1, 6)),
            full((128, 6)), full((1, 128)),
            full((256, 128)), full((1, 256)),
            full((1, 256)), full((1, 1)),
        ],
        out_specs=pl.BlockSpec((_BLOCK, 1), lambda i: (i, 0)),
        out_shape=jax.ShapeDtypeStruct((rows, 1), jnp.float32),
    )(a, W1, b1[None], We1, be1[None], We2, be2[None],
      Wd1, bd1[None], Wd2, bd2[None], Wo, bo[None])
    return out[:, 0]


# speedup vs baseline: 2.2020x; 1.1971x over previous
import jax
import jax.numpy as jnp
from jax.experimental import pallas as pl

_K = 32
_BLOCK = 2048

_DN = (((1,), (1,)), ((), ()))


def _mlp_kernel(a_ref, w1_ref, b1_ref, we1_ref, be1_ref, we2_ref, be2_ref,
                wd1_ref, bd1_ref, wd2_ref, bd2_ref, wo_ref, bo_ref, out_ref):
    f32 = jnp.float32

    def lin(v, w_ref, b_ref):
        return jax.lax.dot_general(v, w_ref[...], _DN,
                                   preferred_element_type=f32) + b_ref[...][None, :]

    a = a_ref[...]
    h = jnp.tanh(lin(a, w1_ref, b1_ref))
    h = jnp.tanh(lin(h, we1_ref, be1_ref))
    e = jnp.tanh(lin(h, we2_ref, be2_ref))
    h = jnp.tanh(lin(e, wd1_ref, bd1_ref))
    d = lin(h, wd2_ref, bd2_ref)
    t = jnp.tanh(d)
    o = jax.lax.dot_general(wo_ref[...], t, _DN, preferred_element_type=f32)
    out_ref[...] = jax.nn.sigmoid(o + bo_ref[...][None, :])


def kernel(x, edge_index, edge_attr, W_ih, W_hh, b_ih, b_hh, W1, b1, Wg, bg,
           We1, be1, We2, be2, Wd1, bd1, Wd2, bd2, Wo, bo):
    a = edge_attr.reshape(-1, _K)
    rows = a.shape[0]

    def full(shape):
        return pl.BlockSpec(shape, lambda i: (0,) * len(shape))

    out = pl.pallas_call(
        _mlp_kernel,
        grid=(pl.cdiv(rows, _BLOCK),),
        in_specs=[
            pl.BlockSpec((_BLOCK, _K), lambda i: (i, 0)),
            full((256, _K)), full((256,)),
            full((128, 256)), full((128,)),
            full((6, 128)), full((6,)),
            full((128, 6)), full((128,)),
            full((256, 128)), full((256,)),
            full((1, 256)), full((1,)),
        ],
        out_specs=pl.BlockSpec((1, _BLOCK), lambda i: (0, i)),
        out_shape=jax.ShapeDtypeStruct((1, rows), jnp.float32),
    )(a, W1, b1, We1, be1, We2, be2,
      Wd1, bd1, Wd2, bd2, Wo, bo)
    return out[0]


# speedup vs baseline: 2.3673x; 1.0751x over previous
import jax
import jax.numpy as jnp
from jax.experimental import pallas as pl

_K = 32
_BLOCK = 10000

_DN = (((1,), (1,)), ((), ()))


def _mlp_kernel(a_ref, w1_ref, b1_ref, we1_ref, be1_ref, we2_ref, be2_ref,
                wd1_ref, bd1_ref, wd2_ref, bd2_ref, wo_ref, bo_ref, out_ref):
    f32 = jnp.float32

    def lin(v, w_ref, b_ref):
        return jax.lax.dot_general(v, w_ref[...], _DN,
                                   preferred_element_type=f32) + b_ref[...][None, :]

    a = a_ref[...]
    h = jnp.tanh(lin(a, w1_ref, b1_ref))
    h = jnp.tanh(lin(h, we1_ref, be1_ref))
    e = jnp.tanh(lin(h, we2_ref, be2_ref))
    h = jnp.tanh(lin(e, wd1_ref, bd1_ref))
    d = lin(h, wd2_ref, bd2_ref)
    t = jnp.tanh(d)
    o = jax.lax.dot_general(wo_ref[...], t, _DN, preferred_element_type=f32)
    out_ref[...] = jax.nn.sigmoid(o + bo_ref[...][None, :]).reshape(-1)


def kernel(x, edge_index, edge_attr, W_ih, W_hh, b_ih, b_hh, W1, b1, Wg, bg,
           We1, be1, We2, be2, Wd1, bd1, Wd2, bd2, Wo, bo):
    a = edge_attr.reshape(-1, _K)
    rows = a.shape[0]

    def full(shape):
        return pl.BlockSpec(shape, lambda i: (0,) * len(shape))

    out = pl.pallas_call(
        _mlp_kernel,
        grid=(pl.cdiv(rows, _BLOCK),),
        in_specs=[
            pl.BlockSpec((_BLOCK, _K), lambda i: (i, 0)),
            full((256, _K)), full((256,)),
            full((128, 256)), full((128,)),
            full((6, 128)), full((6,)),
            full((128, 6)), full((128,)),
            full((256, 128)), full((256,)),
            full((1, 256)), full((1,)),
        ],
        out_specs=pl.BlockSpec((_BLOCK,), lambda i: (i,)),
        out_shape=jax.ShapeDtypeStruct((rows,), jnp.float32),
    )(a, W1, b1, We1, be1, We2, be2,
      Wd1, bd1, Wd2, bd2, Wo, bo)
    return out


# speedup vs baseline: 4.8825x; 2.0625x over previous
import jax
import jax.numpy as jnp
from jax.experimental import pallas as pl
from jax.experimental.pallas import tpu as pltpu

_DN = (((1,), (1,)), ((), ()))
_DN0 = (((1,), (0,)), ((), ()))


def _mlp_kernel(a_ref, w1t_ref, b1_ref, we1_ref, be1_ref, we2_ref, be2_ref,
                wd1t_ref, bd1_ref, wd2_ref, bd2_ref, wo_ref, bo_ref, out_ref,
                a_scr):
    f32 = jnp.float32
    ah = a_ref[...]
    for p in range(4):
        a_scr[p::4, :] = ah[:, 32 * p:32 * (p + 1)]
    a = a_scr[...]
    h = jnp.tanh(jax.lax.dot_general(a, w1t_ref[...], _DN0,
                                     preferred_element_type=f32)
                 + b1_ref[...][None, :])
    h = jnp.tanh(jax.lax.dot_general(h, we1_ref[...], _DN,
                                     preferred_element_type=f32)
                 + be1_ref[...][None, :])
    e = jnp.tanh(jax.lax.dot_general(h, we2_ref[...], _DN,
                                     preferred_element_type=f32)
                 + be2_ref[...][None, :])
    h = jnp.tanh(jax.lax.dot_general(e, wd1t_ref[...], _DN0,
                                     preferred_element_type=f32)
                 + bd1_ref[...][None, :])
    d = (jax.lax.dot_general(h, wd2_ref[...], _DN,
                             preferred_element_type=f32)
         + bd2_ref[...][None, :])
    t = jnp.tanh(d)
    o = jax.lax.dot_general(wo_ref[...], t, _DN, preferred_element_type=f32)
    n_out = out_ref.shape[0]
    out_ref[...] = jax.nn.sigmoid(o + bo_ref[...][None, :])[:, :n_out].reshape(-1)


def kernel(x, edge_index, edge_attr, W_ih, W_hh, b_ih, b_hh, W1, b1, Wg, bg,
           We1, be1, We2, be2, Wd1, bd1, Wd2, bd2, Wo, bo):
    rows = edge_attr.shape[0] // 32
    ep = jnp.pad(edge_attr, ((0, 512), (0, 0)))
    r128 = ep.shape[0] // 128
    a128 = ep.reshape(r128, 128)

    def full(shape):
        return pl.BlockSpec(shape, lambda: (0,) * len(shape))

    out = pl.pallas_call(
        _mlp_kernel,
        in_specs=[
            full((r128, 128)),
            full((32, 256)), full((256,)),
            full((128, 256)), full((128,)),
            full((6, 128)), full((6,)),
            full((6, 128)), full((128,)),
            full((256, 128)), full((256,)),
            full((1, 256)), full((1,)),
        ],
        out_specs=pl.BlockSpec((rows,), lambda: (0,)),
        out_shape=jax.ShapeDtypeStruct((rows,), jnp.float32),
        scratch_shapes=[pltpu.VMEM((r128 * 4, 32), jnp.float32)],
    )(a128, W1.T, b1, We1, be1, We2, be2,
      Wd1.T, bd1, Wd2, bd2, Wo, bo)
    return out
